# Initial kernel scaffold; baseline (speedup 1.0000x reference)
import jax
import jax.numpy as jnp
from jax.experimental import pallas as pl


def kernel(inputs, embed, W1, b1, W2, b2):
    raise NotImplementedError("write your pallas kernel here")


# speedup vs baseline: 8.0929x; 8.0929x over previous
import functools

import jax
import jax.numpy as jnp
from jax import lax
from jax.experimental import pallas as pl
from jax.experimental.pallas import tpu as pltpu
from jax.experimental.pallas import tpu_sc as plsc

N_VOCAB = 123
HIDDEN = 1000
N_OUT = 123

_NC = 2
_NS = 16
_NW = _NC * _NS
_CH = 128
_NCHUNK = 20
_DPAD = 128


def _table_body(emb_ref, w1_ref, b1_ref, w2_ref, b2_ref, out_ref):
    h = jnp.maximum(emb_ref[...], 0.0)
    h = jnp.dot(h, w1_ref[...], preferred_element_type=jnp.float32) + b1_ref[...]
    h = jnp.maximum(h, 0.0)
    out_ref[...] = (
        jnp.dot(h, w2_ref[...], preferred_element_type=jnp.float32) + b2_ref[...]
    )


_table_call = pl.pallas_call(
    _table_body,
    out_shape=jax.ShapeDtypeStruct((N_VOCAB, N_OUT), jnp.float32),
)


@functools.cache
def _make_gather_call():
    mesh = plsc.VectorSubcoreMesh(core_axis_name="c", subcore_axis_name="s")

    @functools.partial(
        pl.kernel,
        mesh=mesh,
        out_type=jax.ShapeDtypeStruct((_NW, _NCHUNK, _CH, _DPAD), jnp.float32),
        scratch_types=[
            pltpu.VMEM((_NCHUNK, _CH), jnp.int32),
            pltpu.VMEM((_CH, _DPAD), jnp.float32),
            pltpu.VMEM((_CH, _DPAD), jnp.float32),
            pltpu.SemaphoreType.DMA,
            pltpu.SemaphoreType.DMA,
        ],
    )
    def _gather_call(idx_hbm, table_hbm, out_hbm, idx_v, rows0, rows1, sem0, sem1):
        wid = lax.axis_index("s") * _NC + lax.axis_index("c")
        pltpu.sync_copy(idx_hbm.at[wid], idx_v)
        bufs = (rows0, rows1)
        sems = (sem0, sem1)
        copies = [None, None]
        copies[0] = pltpu.async_copy(table_hbm.at[idx_v.at[0]], bufs[0], sems[0])
        for j in range(_NCHUNK):
            cur = j % 2
            nxt = 1 - cur
            if j + 1 < _NCHUNK:
                copies[nxt] = pltpu.async_copy(
                    table_hbm.at[idx_v.at[j + 1]], bufs[nxt], sems[nxt]
                )
            copies[cur].wait()
            pltpu.sync_copy(bufs[cur], out_hbm.at[wid, j])

    return _gather_call


def kernel(inputs, embed, W1, b1, W2, b2):
    table = _table_call(
        embed, W1, b1.reshape(1, HIDDEN), W2, b2.reshape(1, N_OUT)
    )
    table = jnp.pad(table, ((0, _DPAD - N_VOCAB), (0, _DPAD - N_OUT)))
    B, L = inputs.shape
    idx = inputs.astype(jnp.int32).reshape(_NW, _NCHUNK, _CH)
    out = _make_gather_call()(idx, table)
    return out[..., :N_OUT].reshape(B, L, N_OUT)


# speedup vs baseline: 9.3964x; 1.1611x over previous
import functools

import jax
import jax.numpy as jnp
from jax import lax
from jax.experimental import pallas as pl
from jax.experimental.pallas import tpu as pltpu
from jax.experimental.pallas import tpu_sc as plsc

N_VOCAB = 123
HIDDEN = 1000
N_OUT = 123

_NC = 2
_NS = 16
_NW = _NC * _NS
_CH = 128
_NCHUNK = 20
_SUP = 2
_NSUP = _NCHUNK // _SUP
_NBUF = 3
_DPAD = 128


def _table_body(emb_ref, w1_ref, b1_ref, w2_ref, b2_ref, out_ref):
    h = jnp.maximum(emb_ref[...], 0.0)
    h = jnp.dot(h, w1_ref[...], preferred_element_type=jnp.float32) + b1_ref[...]
    h = jnp.maximum(h, 0.0)
    t = jnp.dot(h, w2_ref[...], preferred_element_type=jnp.float32) + b2_ref[...]
    out_ref[...] = jnp.pad(
        t, ((0, _DPAD - N_VOCAB), (0, _DPAD - N_OUT))
    )


_table_call = pl.pallas_call(
    _table_body,
    out_shape=jax.ShapeDtypeStruct((_DPAD, _DPAD), jnp.float32),
)


@functools.cache
def _make_gather_call():
    mesh = plsc.VectorSubcoreMesh(core_axis_name="c", subcore_axis_name="s")

    @functools.partial(
        pl.kernel,
        mesh=mesh,
        out_type=jax.ShapeDtypeStruct(
            (_NW, _NSUP, _SUP * _CH, _DPAD), jnp.float32
        ),
        scratch_types=[
            pltpu.VMEM((_NCHUNK, _CH), jnp.int32),
            pltpu.VMEM((_NBUF, _SUP * _CH, _DPAD), jnp.float32),
            pltpu.SemaphoreType.DMA,
            pltpu.SemaphoreType.DMA,
            pltpu.SemaphoreType.DMA,
            pltpu.SemaphoreType.DMA,
        ],
    )
    def _gather_call(idx_hbm, table_hbm, out_hbm, idx_v, rows, g0, g1, g2, wsem):
        wid = lax.axis_index("s") * _NC + lax.axis_index("c")
        pltpu.sync_copy(idx_hbm.at[wid], idx_v)
        gsems = (g0, g1, g2)
        wcopies = [None] * _NSUP
        gcopies = [None] * _NBUF
        for s in range(_NSUP):
            buf = s % _NBUF
            if s >= _NBUF:
                wcopies[s - _NBUF].wait()
            gcopies[buf] = [
                pltpu.async_copy(
                    table_hbm.at[idx_v.at[s * _SUP + k]],
                    rows.at[buf, pl.ds(k * _CH, _CH)],
                    gsems[buf],
                )
                for k in range(_SUP)
            ]
            for cp in gcopies[buf]:
                cp.wait()
            wcopies[s] = pltpu.async_copy(rows.at[buf], out_hbm.at[wid, s], wsem)
        for s in range(_NSUP - _NBUF, _NSUP):
            wcopies[s].wait()

    return _gather_call


def kernel(inputs, embed, W1, b1, W2, b2):
    table = _table_call(
        embed, W1, b1.reshape(1, HIDDEN), W2, b2.reshape(1, N_OUT)
    )
    B, L = inputs.shape
    idx = inputs.astype(jnp.int32).reshape(_NW, _NCHUNK, _CH)
    out = _make_gather_call()(idx, table)
    return out.reshape(B * L, _DPAD)[:, :N_OUT].reshape(B, L, N_OUT)


# speedup vs baseline: 13.8703x; 1.4761x over previous
import functools

import jax
import jax.numpy as jnp
from jax import lax
from jax.experimental import pallas as pl
from jax.experimental.pallas import tpu as pltpu
from jax.experimental.pallas import tpu_sc as plsc

N_VOCAB = 123
HIDDEN = 1000
N_OUT = 123

_NC = 2
_NS = 16
_NW = _NC * _NS
_CH = 128
_NCHUNK = 20
_SUP = 2
_NSUP = _NCHUNK // _SUP
_NBUF = 3
_DPAD = 128


def _table_body(emb_ref, w1_ref, b1_ref, w2_ref, b2_ref, out_ref):
    h = jnp.maximum(emb_ref[...], 0.0)
    h = jnp.dot(h, w1_ref[...], preferred_element_type=jnp.float32) + b1_ref[...]
    h = jnp.maximum(h, 0.0)
    t = jnp.dot(h, w2_ref[...], preferred_element_type=jnp.float32) + b2_ref[...]
    out_ref[...] = jnp.pad(
        t, ((0, _DPAD - N_VOCAB), (0, _DPAD - N_OUT))
    )


_table_call = pl.pallas_call(
    _table_body,
    out_shape=jax.ShapeDtypeStruct((_DPAD, _DPAD), jnp.float32),
)


@functools.cache
def _make_gather_call():
    mesh = plsc.VectorSubcoreMesh(core_axis_name="c", subcore_axis_name="s")

    @functools.partial(
        pl.kernel,
        mesh=mesh,
        out_type=jax.ShapeDtypeStruct(
            (_NW, _NSUP, _SUP * _CH, _DPAD), jnp.float32
        ),
        scratch_types=[
            pltpu.VMEM((_NCHUNK, _CH), jnp.int32),
            pltpu.VMEM((_NBUF, _SUP * _CH, _DPAD), jnp.float32),
            pltpu.VMEM_SHARED((_DPAD, _DPAD), jnp.float32),
            pltpu.SemaphoreType.DMA,
            pltpu.SemaphoreType.DMA,
            pltpu.SemaphoreType.DMA,
            pltpu.SemaphoreType.DMA,
        ],
    )
    def _gather_call(
        idx_hbm, table_hbm, out_hbm, idx_v, rows, table_sp, g0, g1, g2, wsem
    ):
        sid = lax.axis_index("s")
        wid = sid * _NC + lax.axis_index("c")
        @pl.when(sid == 0)
        def _():
            pltpu.sync_copy(table_hbm, table_sp)

        pltpu.sync_copy(idx_hbm.at[wid], idx_v)
        plsc.subcore_barrier()
        gsems = (g0, g1, g2)
        wcopies = [None] * _NSUP
        gcopies = [None] * _NBUF
        for s in range(_NSUP):
            buf = s % _NBUF
            if s >= _NBUF:
                wcopies[s - _NBUF].wait()
            gcopies[buf] = [
                pltpu.async_copy(
                    table_sp.at[idx_v.at[s * _SUP + k]],
                    rows.at[buf, pl.ds(k * _CH, _CH)],
                    gsems[buf],
                )
                for k in range(_SUP)
            ]
            for cp in gcopies[buf]:
                cp.wait()
            wcopies[s] = pltpu.async_copy(rows.at[buf], out_hbm.at[wid, s], wsem)
        for s in range(_NSUP - _NBUF, _NSUP):
            wcopies[s].wait()

    return _gather_call


def kernel(inputs, embed, W1, b1, W2, b2):
    table = _table_call(
        embed, W1, b1.reshape(1, HIDDEN), W2, b2.reshape(1, N_OUT)
    )
    B, L = inputs.shape
    idx = inputs.astype(jnp.int32).reshape(_NW, _NCHUNK, _CH)
    out = _make_gather_call()(idx, table)
    return out.reshape(B * L, _DPAD)[:, :N_OUT].reshape(B, L, N_OUT)


# speedup vs baseline: 13.9046x; 1.0025x over previous
import functools

import jax
import jax.numpy as jnp
from jax import lax
from jax.experimental import pallas as pl
from jax.experimental.pallas import tpu as pltpu
from jax.experimental.pallas import tpu_sc as plsc

N_VOCAB = 123
HIDDEN = 1000
N_OUT = 123

_NC = 2
_NS = 16
_NW = _NC * _NS
_CH = 128
_NCHUNK = 20
_SUP = 2
_NSUP = _NCHUNK // _SUP
_NBUF = 3
_DPAD = 128


def _table_body(emb_ref, w1_ref, b1_ref, w2_ref, b2_ref, out_ref):
    h = jnp.maximum(emb_ref[...], 0.0)
    h = jnp.dot(h, w1_ref[...], preferred_element_type=jnp.float32) + b1_ref[...]
    h = jnp.maximum(h, 0.0)
    t = jnp.dot(h, w2_ref[...], preferred_element_type=jnp.float32) + b2_ref[...]
    out_ref[...] = jnp.pad(
        t, ((0, _DPAD - N_VOCAB), (0, _DPAD - N_OUT))
    )


_table_call = pl.pallas_call(
    _table_body,
    out_shape=jax.ShapeDtypeStruct((_DPAD, _DPAD), jnp.float32),
)


@functools.cache
def _make_gather_call():
    mesh = plsc.VectorSubcoreMesh(core_axis_name="c", subcore_axis_name="s")

    @functools.partial(
        pl.kernel,
        mesh=mesh,
        out_type=jax.ShapeDtypeStruct((_NW * _NCHUNK * _CH, _DPAD), jnp.float32),
        scratch_types=[
            pltpu.VMEM((_NCHUNK, _CH), jnp.int32),
            pltpu.VMEM((_NBUF, _SUP * _CH, _DPAD), jnp.float32),
            pltpu.VMEM_SHARED((_DPAD, _DPAD), jnp.float32),
            pltpu.SemaphoreType.DMA,
            pltpu.SemaphoreType.DMA,
            pltpu.SemaphoreType.DMA,
            pltpu.SemaphoreType.DMA,
        ],
    )
    def _gather_call(
        idx_hbm, table_hbm, out_hbm, idx_v, rows, table_sp, g0, g1, g2, wsem
    ):
        sid = lax.axis_index("s")
        wid = sid * _NC + lax.axis_index("c")
        @pl.when(sid == 0)
        def _():
            pltpu.sync_copy(table_hbm, table_sp)

        pltpu.sync_copy(idx_hbm.at[wid], idx_v)
        plsc.subcore_barrier()
        gsems = (g0, g1, g2)
        wcopies = [None] * _NSUP
        gcopies = [None] * _NBUF
        for s in range(_NSUP):
            buf = s % _NBUF
            if s >= _NBUF:
                wcopies[s - _NBUF].wait()
            gcopies[buf] = [
                pltpu.async_copy(
                    table_sp.at[idx_v.at[s * _SUP + k]],
                    rows.at[buf, pl.ds(k * _CH, _CH)],
                    gsems[buf],
                )
                for k in range(_SUP)
            ]
            for cp in gcopies[buf]:
                cp.wait()
            wcopies[s] = pltpu.async_copy(
                rows.at[buf],
                out_hbm.at[pl.ds(wid * _NCHUNK * _CH + s * _SUP * _CH, _SUP * _CH)],
                wsem,
            )
        for s in range(_NSUP - _NBUF, _NSUP):
            wcopies[s].wait()

    return _gather_call


def kernel(inputs, embed, W1, b1, W2, b2):
    table = _table_call(
        embed, W1, b1.reshape(1, HIDDEN), W2, b2.reshape(1, N_OUT)
    )
    B, L = inputs.shape
    idx = inputs.astype(jnp.int32).reshape(_NW, _NCHUNK, _CH)
    out = _make_gather_call()(idx, table)
    return out[:, :N_OUT].reshape(B, L, N_OUT)


# speedup vs baseline: 14.7782x; 1.0628x over previous
import functools

import jax
import jax.numpy as jnp
from jax import lax
from jax.experimental import pallas as pl
from jax.experimental.pallas import tpu as pltpu
from jax.experimental.pallas import tpu_sc as plsc

N_VOCAB = 123
HIDDEN = 1000
N_OUT = 123

_NC = 2
_NS = 16
_NW = _NC * _NS
_CH = 128
_NCHUNK = 20
_SUP = 2
_NSUP = _NCHUNK // _SUP
_NBUF = 3
_DPAD = 128


def _table_body(emb_ref, w1_ref, b1_ref, w2_ref, b2_ref, out_ref):
    h = jnp.maximum(emb_ref[...], 0.0)
    h = jnp.dot(h, w1_ref[...], preferred_element_type=jnp.float32) + b1_ref[...]
    h = jnp.maximum(h, 0.0)
    t = jnp.dot(h, w2_ref[...], preferred_element_type=jnp.float32) + b2_ref[...]
    out_ref[...] = jnp.pad(
        t, ((0, _DPAD - N_VOCAB), (0, _DPAD - N_OUT))
    )


_table_call = pl.pallas_call(
    _table_body,
    out_shape=jax.ShapeDtypeStruct((_DPAD, _DPAD), jnp.float32),
)


def _relayout_body(in_ref, out_ref):
    x = in_ref[...]
    out_ref[...] = x[:, :N_OUT].reshape(out_ref.shape)


@functools.cache
def _make_gather_call():
    mesh = plsc.VectorSubcoreMesh(core_axis_name="c", subcore_axis_name="s")

    @functools.partial(
        pl.kernel,
        mesh=mesh,
        out_type=jax.ShapeDtypeStruct((_NW * _NCHUNK * _CH, _DPAD), jnp.float32),
        scratch_types=[
            pltpu.VMEM((_NCHUNK, _CH), jnp.int32),
            pltpu.VMEM((_NBUF, _SUP * _CH, _DPAD), jnp.float32),
            pltpu.VMEM_SHARED((_DPAD, _DPAD), jnp.float32),
            pltpu.SemaphoreType.DMA,
            pltpu.SemaphoreType.DMA,
            pltpu.SemaphoreType.DMA,
            pltpu.SemaphoreType.DMA,
        ],
    )
    def _gather_call(
        idx_hbm, table_hbm, out_hbm, idx_v, rows, table_sp, g0, g1, g2, wsem
    ):
        sid = lax.axis_index("s")
        wid = sid * _NC + lax.axis_index("c")
        @pl.when(sid == 0)
        def _():
            pltpu.sync_copy(table_hbm, table_sp)

        pltpu.sync_copy(idx_hbm.at[wid], idx_v)
        plsc.subcore_barrier()
        gsems = (g0, g1, g2)
        wcopies = [None] * _NSUP
        gcopies = [None] * _NBUF
        for s in range(_NSUP):
            buf = s % _NBUF
            if s >= _NBUF:
                wcopies[s - _NBUF].wait()
            gcopies[buf] = [
                pltpu.async_copy(
                    table_sp.at[idx_v.at[s * _SUP + k]],
                    rows.at[buf, pl.ds(k * _CH, _CH)],
                    gsems[buf],
                )
                for k in range(_SUP)
            ]
            for cp in gcopies[buf]:
                cp.wait()
            wcopies[s] = pltpu.async_copy(
                rows.at[buf],
                out_hbm.at[pl.ds(wid * _NCHUNK * _CH + s * _SUP * _CH, _SUP * _CH)],
                wsem,
            )
        for s in range(_NSUP - _NBUF, _NSUP):
            wcopies[s].wait()

    return _gather_call


def kernel(inputs, embed, W1, b1, W2, b2):
    table = _table_call(
        embed, W1, b1.reshape(1, HIDDEN), W2, b2.reshape(1, N_OUT)
    )
    B, L = inputs.shape
    idx = inputs.astype(jnp.int32).reshape(_NW, _NCHUNK, _CH)
    out = _make_gather_call()(idx, table)
    bpw = B // _NW
    return pl.pallas_call(
        _relayout_body,
        grid=(_NW,),
        in_specs=[
            pl.BlockSpec((bpw * L, _DPAD), lambda i: (i, 0)),
        ],
        out_specs=pl.BlockSpec((bpw, L, N_OUT), lambda i: (i, 0, 0)),
        out_shape=jax.ShapeDtypeStruct((B, L, N_OUT), jnp.float32),
    )(out)


# speedup vs baseline: 20.1630x; 1.3644x over previous
import functools

import jax
import jax.numpy as jnp
from jax import lax
from jax.experimental import pallas as pl
from jax.experimental.pallas import tpu as pltpu
from jax.experimental.pallas import tpu_sc as plsc

N_VOCAB = 123
HIDDEN = 1000
N_OUT = 123
B = 4096
L = 20
_LP = 24

_NC = 2
_NS = 16
_NW = _NC * _NS
_CH = 128
_NCHUNK = B * _LP // (_NW * _CH)
_SUP = 2
_NSUP = _NCHUNK // _SUP
_NBUF = 3
_DPAD = 128


def _table_body(emb_ref, w1_ref, b1_ref, w2_ref, b2_ref, out_ref):
    h = jnp.maximum(emb_ref[...], 0.0)
    h = jnp.dot(h, w1_ref[...], preferred_element_type=jnp.float32) + b1_ref[...]
    h = jnp.maximum(h, 0.0)
    t = jnp.dot(h, w2_ref[...], preferred_element_type=jnp.float32) + b2_ref[...]
    out_ref[...] = jnp.pad(
        t, ((0, _DPAD - N_VOCAB), (0, _DPAD - N_OUT))
    )


_table_call = pl.pallas_call(
    _table_body,
    out_shape=jax.ShapeDtypeStruct((_DPAD, _DPAD), jnp.float32),
)


@functools.cache
def _make_gather_call():
    mesh = plsc.VectorSubcoreMesh(core_axis_name="c", subcore_axis_name="s")

    @functools.partial(
        pl.kernel,
        mesh=mesh,
        out_type=jax.ShapeDtypeStruct((B, _LP, _DPAD), jnp.float32),
        scratch_types=[
            pltpu.VMEM((_NCHUNK, _CH), jnp.int32),
            pltpu.VMEM((_NBUF, _SUP * _CH, _DPAD), jnp.float32),
            pltpu.VMEM_SHARED((_DPAD, _DPAD), jnp.float32),
            pltpu.SemaphoreType.DMA,
            pltpu.SemaphoreType.DMA,
            pltpu.SemaphoreType.DMA,
            pltpu.SemaphoreType.DMA,
        ],
    )
    def _gather_call(
        idx_hbm, table_hbm, out_hbm, idx_v, rows, table_sp, g0, g1, g2, wsem
    ):
        sid = lax.axis_index("s")
        wid = sid * _NC + lax.axis_index("c")
        @pl.when(sid == 0)
        def _():
            pltpu.sync_copy(table_hbm, table_sp)

        pltpu.sync_copy(idx_hbm.at[wid], idx_v)
        plsc.subcore_barrier()
        out_flat = out_hbm.reshape(B * _LP, _DPAD)
        gsems = (g0, g1, g2)
        wcopies = [None] * _NSUP
        for s in range(_NSUP):
            buf = s % _NBUF
            if s >= _NBUF:
                wcopies[s - _NBUF].wait()
            gcopies = [
                pltpu.async_copy(
                    table_sp.at[idx_v.at[s * _SUP + k]],
                    rows.at[buf, pl.ds(k * _CH, _CH)],
                    gsems[buf],
                )
                for k in range(_SUP)
            ]
            for cp in gcopies:
                cp.wait()
            wcopies[s] = pltpu.async_copy(
                rows.at[buf],
                out_flat.at[
                    pl.ds(wid * _NCHUNK * _CH + s * _SUP * _CH, _SUP * _CH)
                ],
                wsem,
            )
        for s in range(_NSUP - _NBUF, _NSUP):
            wcopies[s].wait()

    return _gather_call


def kernel(inputs, embed, W1, b1, W2, b2):
    table = _table_call(
        embed, W1, b1.reshape(1, HIDDEN), W2, b2.reshape(1, N_OUT)
    )
    idx = jnp.pad(inputs.astype(jnp.int32), ((0, 0), (0, _LP - L)))
    idx = idx.reshape(_NW, _NCHUNK, _CH)
    out = _make_gather_call()(idx, table)
    return out[:, :L, :N_OUT]
